# Initial kernel scaffold; baseline (speedup 1.0000x reference)
import jax
import jax.numpy as jnp
from jax.experimental import pallas as pl


def kernel(pos_u, pos_v, neg_v, batch_size, u_weight, v_weight):
    raise NotImplementedError("write your pallas kernel here")


# speedup vs baseline: 1.5594x; 1.5594x over previous
import functools

import jax
import jax.numpy as jnp
from jax import lax
from jax.experimental import pallas as pl
from jax.experimental.pallas import tpu as pltpu
from jax.experimental.pallas import tpu_sc as plsc

_B = 16384
_D = 64
_K = 5
_NC = 2
_NS = 16
_L = 16
_NW = _NC * _NS
_C = 128
_CHUNKS = _B // (_NW * _C)
_GROUPS = _C // _L


def _sc_body(pos_u_hbm, pos_v_hbm, neg_hbm, uw_hbm, vw_hbm,
             pos_out, neg_out,
             idx_u, idx_v, idx_n, rows_u, rows_v, rows_n,
             acc_pos, acc_neg, sem):
    wid = lax.axis_index("s") * _NC + lax.axis_index("c")
    iota = lax.iota(jnp.int32, _L)

    def chunk_body(c, carry):
        base = pl.multiple_of(wid * (_CHUNKS * _C) + c * _C, _C)
        chunk_id = wid * _CHUNKS + c
        pltpu.sync_copy(pos_u_hbm.at[pl.ds(base, _C)], idx_u)
        pltpu.sync_copy(pos_v_hbm.at[pl.ds(base, _C)], idx_v)
        for j in range(_K):
            off = pl.multiple_of(base * _K + j * _C, _C)
            pltpu.sync_copy(neg_hbm.at[pl.ds(off, _C)], idx_n.at[j])
        cps = [pltpu.async_copy(uw_hbm.at[idx_u], rows_u, sem),
               pltpu.async_copy(vw_hbm.at[idx_v], rows_v, sem)]
        for j in range(_K):
            cps.append(pltpu.async_copy(vw_hbm.at[idx_n.at[j]],
                                        rows_n.at[pl.ds(j * _C, _C)], sem))
        for cp in cps:
            cp.wait()

        def group_body(g, carry2):
            rowp = iota + g * _L
            nrows = [iota * _K + (g * (_L * _K) + k) for k in range(_K)]

            def d_body(dd, accs):
                colv = jnp.zeros((_L,), jnp.int32) + dd
                u_d = plsc.load_gather(rows_u, [rowp, colv])
                v_d = plsc.load_gather(rows_v, [rowp, colv])
                new = [accs[0] + u_d * v_d]
                for k in range(_K):
                    n_d = plsc.load_gather(rows_n, [nrows[k], colv])
                    new.append(accs[k + 1] + u_d * n_d)
                return tuple(new)

            zero = jnp.zeros((_L,), jnp.float32)
            accs = lax.fori_loop(0, _D, d_body, (zero,) * (_K + 1))
            acc_pos[pl.ds(g * _L, _L)] = accs[0]
            for k in range(_K):
                acc_neg[k, pl.ds(g * _L, _L)] = accs[k + 1]
            return carry2

        lax.fori_loop(0, _GROUPS, group_body, 0)
        pltpu.sync_copy(acc_pos, pos_out.at[pl.ds(base, _C)])
        pltpu.sync_copy(acc_neg, neg_out.at[chunk_id])
        return carry

    lax.fori_loop(0, _CHUNKS, chunk_body, 0)


@functools.partial(jax.jit, static_argnames=("interpret",))
def _sc_logits(pos_u, pos_v, neg_flat, u_weight, v_weight, interpret=False):
    mesh = plsc.VectorSubcoreMesh(core_axis_name="c", subcore_axis_name="s",
                                  num_cores=_NC, num_subcores=_NS)
    kfn = pl.kernel(
        _sc_body,
        out_type=(jax.ShapeDtypeStruct((_B,), jnp.float32),
                  jax.ShapeDtypeStruct((_B // _C, _K, _C), jnp.float32)),
        mesh=mesh,
        scratch_types=[
            pltpu.VMEM((_C,), jnp.int32),
            pltpu.VMEM((_C,), jnp.int32),
            pltpu.VMEM((_K, _C), jnp.int32),
            pltpu.VMEM((_C, _D), jnp.float32),
            pltpu.VMEM((_C, _D), jnp.float32),
            pltpu.VMEM((_K * _C, _D), jnp.float32),
            pltpu.VMEM((_C,), jnp.float32),
            pltpu.VMEM((_K, _C), jnp.float32),
            pltpu.SemaphoreType.DMA,
        ],
        compiler_params=pltpu.CompilerParams(needs_layout_passes=False,
                                             use_tc_tiling_on_sc=False),
        interpret=interpret,
    )
    return kfn(pos_u, pos_v, neg_flat, u_weight, v_weight)


def _loss_body(pos_ref, neg_ref, out_ref):
    total = (jnp.sum(jax.nn.log_sigmoid(pos_ref[...]))
             + jnp.sum(jax.nn.log_sigmoid(-neg_ref[...])))
    out_ref[...] = jnp.reshape(total, (1, 1))


def _tc_loss(pos_logits, neg_logits, interpret=False):
    return pl.pallas_call(
        _loss_body,
        out_shape=jax.ShapeDtypeStruct((1, 1), jnp.float32),
        interpret=interpret,
    )(pos_logits, neg_logits)


def kernel(pos_u, pos_v, neg_v, batch_size, u_weight, v_weight):
    pos_u = pos_u.astype(jnp.int32)
    pos_v = pos_v.astype(jnp.int32)
    neg_flat = neg_v.astype(jnp.int32).reshape(-1)
    pos_logits, neg_logits = _sc_logits(pos_u, pos_v, neg_flat,
                                        u_weight, v_weight)
    total = _tc_loss(pos_logits.reshape(_B // _C, _C),
                     neg_logits.reshape(_B // _C * _K, _C))
    return (-total[0, 0] / batch_size).astype(jnp.float32)


# speedup vs baseline: 2.6368x; 1.6909x over previous
import functools

import jax
import jax.numpy as jnp
from jax import lax
from jax.experimental import pallas as pl
from jax.experimental.pallas import tpu as pltpu
from jax.experimental.pallas import tpu_sc as plsc

_B = 16384
_D = 64
_K = 5
_NC = 2
_NS = 16
_L = 16
_NW = _NC * _NS
_EPW = _B // _NW
_ROUNDS = _EPW // _L
_BLK = 8


def _sc_body(pos_u_hbm, pos_v_hbm, neg_hbm, uw3_hbm, vw3_hbm,
             pos_out, neg_out,
             s_iu, s_iv, s_in, ublk, vnblk,
             o_pos, o_neg, sem):
    wid = lax.axis_index("s") * _NC + lax.axis_index("c")
    base = pl.multiple_of(wid * _EPW, _EPW)
    iota = lax.iota(jnp.int32, _L)

    pltpu.sync_copy(pos_u_hbm.at[pl.ds(base, _EPW)], s_iu)
    pltpu.sync_copy(pos_v_hbm.at[pl.ds(base, _EPW)], s_iv)
    pltpu.sync_copy(neg_hbm.at[pl.ds(base * _K, _EPW * _K)], s_in)

    def _scalar(vec, i):
        return jnp.sum(jnp.where(iota == i, vec, 0))

    def round_body(r, carry):
        iu = s_iu[pl.ds(r * _L, _L)]
        iv = s_iv[pl.ds(r * _L, _L)]
        bu_vec = lax.shift_right_logical(iu, 3)
        bv_vec = lax.shift_right_logical(iv, 3)
        cps = []
        for i in range(_L):
            cps.append(pltpu.async_copy(
                uw3_hbm.at[_scalar(bu_vec, i)], ublk.at[i], sem))
            cps.append(pltpu.async_copy(
                vw3_hbm.at[_scalar(bv_vec, i)], vnblk.at[i], sem))
        for j in range(_K):
            cj = s_in[pl.ds(r * (_L * _K) + j * _L, _L)]
            bn_vec = lax.shift_right_logical(cj, 3)
            for i in range(_L):
                cps.append(pltpu.async_copy(
                    vw3_hbm.at[_scalar(bn_vec, i)],
                    vnblk.at[_L + j * _L + i], sem))
        for cp in cps:
            cp.wait()

        usub = lax.bitwise_and(iu, 7)
        vsub = lax.bitwise_and(iv, 7)
        nsubs, nrows = [], []
        for k in range(_K):
            nk = plsc.load_gather(s_in, [iota * _K + (r * (_L * _K) + k)])
            nsubs.append(lax.bitwise_and(nk, 7))
            nrows.append(_L + iota * _K + k)

        def d_body(dd, accs):
            colv = jnp.zeros((_L,), jnp.int32) + dd
            u_d = plsc.load_gather(ublk, [iota, usub, colv])
            v_d = plsc.load_gather(vnblk, [iota, vsub, colv])
            new = [accs[0] + u_d * v_d]
            for k in range(_K):
                n_d = plsc.load_gather(vnblk, [nrows[k], nsubs[k], colv])
                new.append(accs[k + 1] + u_d * n_d)
            return tuple(new)

        zero = jnp.zeros((_L,), jnp.float32)
        accs = lax.fori_loop(0, _D, d_body, (zero,) * (_K + 1))
        o_pos[pl.ds(r * _L, _L)] = accs[0]
        for k in range(_K):
            o_neg[pl.ds(k * _EPW + r * _L, _L)] = accs[k + 1]
        return carry

    lax.fori_loop(0, _ROUNDS, round_body, 0)
    pltpu.sync_copy(o_pos, pos_out.at[pl.ds(base, _EPW)])
    pltpu.sync_copy(o_neg, neg_out.at[pl.ds(base * _K, _EPW * _K)])


@jax.jit
def _sc_logits(pos_u, pos_v, neg_flat, uw3, vw3):
    mesh = plsc.VectorSubcoreMesh(core_axis_name="c", subcore_axis_name="s",
                                  num_cores=_NC, num_subcores=_NS)
    kfn = pl.kernel(
        _sc_body,
        out_type=(jax.ShapeDtypeStruct((_B,), jnp.float32),
                  jax.ShapeDtypeStruct((_B * _K,), jnp.float32)),
        mesh=mesh,
        scratch_types=[
            pltpu.VMEM((_EPW,), jnp.int32),
            pltpu.VMEM((_EPW,), jnp.int32),
            pltpu.VMEM((_EPW * _K,), jnp.int32),
            pltpu.VMEM((_L, _BLK, _D), jnp.float32),
            pltpu.VMEM(((_K + 1) * _L, _BLK, _D), jnp.float32),
            pltpu.VMEM((_EPW,), jnp.float32),
            pltpu.VMEM((_EPW * _K,), jnp.float32),
            pltpu.SemaphoreType.DMA,
        ],
        compiler_params=pltpu.CompilerParams(needs_layout_passes=False),
    )
    return kfn(pos_u, pos_v, neg_flat, uw3, vw3)


def _loss_body(pos_ref, neg_ref, out_ref):
    total = (jnp.sum(jax.nn.log_sigmoid(pos_ref[...]))
             + jnp.sum(jax.nn.log_sigmoid(-neg_ref[...])))
    out_ref[...] = jnp.reshape(total, (1, 1))


def _tc_loss(pos_logits, neg_logits, interpret=False):
    return pl.pallas_call(
        _loss_body,
        out_shape=jax.ShapeDtypeStruct((1, 1), jnp.float32),
        interpret=interpret,
    )(pos_logits, neg_logits)


def kernel(pos_u, pos_v, neg_v, batch_size, u_weight, v_weight):
    pos_u = pos_u.astype(jnp.int32)
    pos_v = pos_v.astype(jnp.int32)
    neg_flat = neg_v.astype(jnp.int32).reshape(-1)
    uw3 = u_weight.reshape(-1, _BLK, _D)
    vw3 = v_weight.reshape(-1, _BLK, _D)
    pos_logits, neg_logits = _sc_logits(pos_u, pos_v, neg_flat, uw3, vw3)
    total = _tc_loss(pos_logits.reshape(_B // 128, 128),
                     neg_logits.reshape(_B * _K // 128, 128))
    return (-total[0, 0] / batch_size).astype(jnp.float32)
